# baseline (speedup 1.0000x reference)
import functools

import jax
import jax.numpy as jnp
from jax import lax
from jax.experimental import pallas as pl
from jax.experimental.pallas import tpu as pltpu
from jax.experimental.pallas import tpu_sc as plsc

NC = 2
NS = 16
LANES = 16
CH = 128
CHG = 512
KTOT = 320
K0 = 224
K1 = KTOT - K0
KMAX = max(K0, K1)
KGMAX = KMAX // 4
GS = 16
NCHUNKS = NS * KTOT


def _sc_partials(n, data_flat, row2, col4, adj4):
  mesh = plsc.VectorSubcoreMesh(core_axis_name="c", subcore_axis_name="s")

  @functools.partial(
      pl.kernel,
      out_type=jax.ShapeDtypeStruct((NC, n), jnp.float32),
      mesh=mesh,
      scratch_types=[
          pltpu.VMEM((KMAX, CH), jnp.int32),
          pltpu.VMEM((KGMAX * CHG,), jnp.int32),
          pltpu.VMEM((KGMAX * CHG,), jnp.float32),
          pltpu.VMEM((KGMAX * CHG,), jnp.float32),
          pltpu.VMEM((n,), jnp.float32),
          pltpu.VMEM_SHARED((n,), jnp.float32),
          pltpu.SemaphoreType.DMA,
          pltpu.SemaphoreType.DMA,
      ],
  )
  def sc_kernel(data_hbm, row_hbm, col_hbm, adj_hbm, out_hbm,
                rowv, offv, adjv, valv, zerov, acc, gsem, ssem):
    c = lax.axis_index("c")
    s = lax.axis_index("s")
    kc = jnp.where(c == 0, K0, K1)
    kg = jnp.where(c == 0, K0 // 4, K1 // 4)
    start = pl.multiple_of(jnp.where(c == 0, s * K0, NS * K0 + s * K1), 8)
    start4 = pl.multiple_of(
        jnp.where(c == 0, s * (K0 // 4), NS * (K0 // 4) + s * (K1 // 4)), 8)

    pltpu.sync_copy(row_hbm.at[pl.ds(start, KMAX)], rowv)
    pltpu.sync_copy(col_hbm.at[pl.ds(start4 * CHG, KGMAX * CHG)], offv)
    pltpu.sync_copy(adj_hbm.at[pl.ds(start4 * CHG, KGMAX * CHG)], adjv)

    @pl.when(s == 0)
    def _():
      def zero_body(i, carry):
        zerov[pl.ds(i * LANES, LANES)] = jnp.zeros((LANES,), jnp.float32)
        return carry
      lax.fori_loop(0, n // LANES, zero_body, 0)
      pltpu.sync_copy(zerov, acc)

    def flat_body(t, carry):
      for j in range(CH // LANES):
        sl = pl.ds(t * CH + j * LANES, LANES)
        offv[sl] = rowv[t, pl.ds(j * LANES, LANES)] * n + offv[sl]
      return carry
    lax.fori_loop(0, kc, flat_body, 0)

    def fire_body(tg, carry):
      sl = pl.ds(tg * CHG, CHG)
      pltpu.make_async_copy(data_hbm.at[offv.at[sl]], valv.at[sl], gsem).start()
      return carry
    lax.fori_loop(0, kg, fire_body, 0)

    def drain_body(tg, carry):
      sl = pl.ds(tg * CHG, CHG)
      pltpu.make_async_copy(data_hbm.at[offv.at[sl]], valv.at[sl], gsem).wait()
      return carry
    lax.fori_loop(0, kg, drain_body, 0)

    def mul_body(t, carry):
      sl = pl.ds(t * LANES, LANES)
      valv[sl] = valv[sl] * adjv[sl]
      return carry
    lax.fori_loop(0, kc * (CH // LANES), mul_body, 0)

    plsc.subcore_barrier()

    def scat_body(b, carry):
      for m in range(GS):
        kk = b * GS + m
        src = valv.at[pl.ds(kk * CH, CH)]
        pltpu.async_copy(src, acc.at[rowv.at[kk]], ssem, add=True)
      for m in range(GS):
        kk = b * GS + m
        src = valv.at[pl.ds(kk * CH, CH)]
        pltpu.make_async_copy(src, acc.at[rowv.at[kk]], ssem).wait()
      return carry
    lax.fori_loop(0, kc // GS, scat_body, 0)

    plsc.subcore_barrier()

    @pl.when(s == 0)
    def _():
      pltpu.sync_copy(acc, out_hbm.at[c])

  return sc_kernel(data_flat, row2, col4, adj4)


def _tc_finish(n, partials):
  def tc_body(p_ref, alpha_ref, araw_ref):
    a = jnp.sum(p_ref[...], axis=0, keepdims=True)
    araw_ref[...] = a
    m = jnp.max(a)
    e = jnp.exp(a - m)
    alpha_ref[...] = e / jnp.sum(e)

  alpha2, araw2 = pl.pallas_call(
      tc_body,
      out_shape=(
          jax.ShapeDtypeStruct((1, n), jnp.float32),
          jax.ShapeDtypeStruct((1, n), jnp.float32),
      ),
  )(partials)
  return alpha2.reshape(n), araw2.reshape(n)


def kernel(data_input, edge_index, adj_values):
  n = data_input.shape[0]
  e = edge_index.shape[1]

  rows_pad = NCHUNKS - K1 + KMAX
  e_pad = rows_pad * CH
  pad = e_pad - e

  row = jnp.pad(edge_index[0], (0, pad))
  col = jnp.pad(edge_index[1], (0, pad))
  adj = jnp.pad(adj_values, (0, pad))

  row2 = row.reshape(rows_pad, CH)
  data_flat = data_input.reshape(-1)

  partials = _sc_partials(n, data_flat, row2, col, adj)
  return _tc_finish(n, partials)
